# baseline (speedup 1.0000x reference)
import functools

import jax
import jax.numpy as jnp
from jax import lax
from jax.experimental import pallas as pl
from jax.experimental.pallas import tpu as pltpu
from jax.experimental.pallas import tpu_sc as plsc

SMOOTH = 1.0
IGNORE_INDEX = 0

_NW = 32
_LANES = 16


def _sc_bincount_body(t_hbm, out_hbm, tloc, tbl, *, chunk, C):
    cid = lax.axis_index("c")
    sid = lax.axis_index("s")
    wid = cid * 16 + sid
    base = wid * chunk

    pltpu.sync_copy(t_hbm.at[pl.ds(base, chunk)], tloc)

    ones16 = jnp.ones((_LANES,), jnp.float32)
    zeros16 = jnp.zeros((_LANES,), jnp.float32)

    def step(i, carry):
        accs = list(carry)
        for u in range(4):
            t = tloc[pl.ds((4 * i + u) * _LANES, _LANES)]
            for c in range(C):
                accs[c] = accs[c] + jnp.where(t == c, ones16, zeros16)
        return tuple(accs)

    accs = lax.fori_loop(0, chunk // (4 * _LANES), step,
                         tuple(zeros16 for _ in range(C)))
    for c in range(C):
        tbl[pl.ds(c * _LANES, _LANES)] = accs[c]
    pltpu.sync_copy(tbl, out_hbm.at[wid])


def _sc_bincount(targets_flat, C):
    total = targets_flat.shape[0]
    chunk = total // _NW
    mesh = plsc.VectorSubcoreMesh(core_axis_name="c", subcore_axis_name="s")
    body = functools.partial(_sc_bincount_body, chunk=chunk, C=C)
    k = pl.kernel(
        body,
        mesh=mesh,
        out_type=jax.ShapeDtypeStruct((_NW, C * _LANES), jnp.float32),
        scratch_types=[
            pltpu.VMEM((chunk,), jnp.int32),
            pltpu.VMEM((C * _LANES,), jnp.float32),
        ],
    )
    return k(targets_flat).reshape(_NW, C, _LANES)


def _stats_body(*refs, B, C, nchunk, tnr):
    x_refs = refs[:C]
    t_ref = refs[C]
    out_ref = refs[C + 1]
    acc_ref = refs[C + 2]

    b = pl.program_id(0)
    n = pl.program_id(1)

    @pl.when((b == 0) & (n == 0))
    def _init():
        acc_ref[...] = jnp.zeros_like(acc_ref)

    rows = [r[0, 0] for r in x_refs]
    tf = t_ref[0, 0].astype(jnp.float32)

    es = [jnp.exp(jnp.minimum(rows[c], 80.0)) for c in range(C)]
    s = es[0]
    for c in range(1, C):
        s = s + es[c]
    r = 1.0 / s

    ones_row = jnp.ones((1, tnr), jnp.float32)

    def mxsum(v):
        return jax.lax.dot(ones_row, v, preferred_element_type=jnp.float32)

    zero = jnp.zeros((), jnp.float32)
    for c in range(C):
        g = es[c] * r
        mask = tf == float(c)
        acc_ref[b, 0, c] += mxsum(jnp.where(mask, g, zero))
        acc_ref[b, 1, c] += mxsum(g)

    @pl.when((b == B - 1) & (n == nchunk - 1))
    def _finish():
        out_ref[...] = jnp.sum(acc_ref[...], axis=3)


def _tc_stats(x, t, B, C, NR, TNR):
    nchunk = NR // TNR
    body = functools.partial(_stats_body, B=B, C=C, nchunk=nchunk, tnr=TNR)

    def xspec(c):
        return pl.BlockSpec((1, 1, TNR, 128), lambda b, n: (b, c, n, 0))

    return pl.pallas_call(
        body,
        grid=(B, nchunk),
        in_specs=[xspec(c) for c in range(C)] + [
            pl.BlockSpec((1, 1, TNR, 128), lambda b, n: (b, 0, n, 0)),
        ],
        out_specs=pl.BlockSpec((B, 2, C, 128), lambda b, n: (0, 0, 0, 0)),
        out_shape=jax.ShapeDtypeStruct((B, 2, C, 128), jnp.float32),
        scratch_shapes=[pltpu.VMEM((B, 2, C, 1, 128), jnp.float32)],
    )(*([x] * C + [t]))


def _combine_body(stats_ref, cnt_ref, out_ref, *, B, C, wpb):
    stats = jnp.sum(stats_ref[...], axis=3)
    cmask = (jax.lax.broadcasted_iota(jnp.int32, (1, C), 1)
             != IGNORE_INDEX).astype(jnp.float32)
    dice_sum = jnp.zeros((), jnp.float32)
    for b in range(B):
        cnt_b = jnp.sum(cnt_ref[pl.ds(b * wpb, wpb)], axis=(0, 2)).reshape(1, C)
        inter = stats[b, 0].reshape(1, C)
        union = stats[b, 1].reshape(1, C) + cnt_b
        dice = (2.0 * inter + SMOOTH) / (union + SMOOTH)
        dice_sum = dice_sum + jnp.sum(dice * cmask)
    out_ref[0] = 1.0 - dice_sum / (B * (C - 1))


def kernel(logits, targets):
    B, C = logits.shape[0], logits.shape[1]
    N = targets.shape[1] * targets.shape[2] * targets.shape[3]
    NR = N // 128
    x = logits.astype(jnp.float32).reshape(B, C, NR, 128)
    t = targets.reshape(B, 1, NR, 128)

    counts = _sc_bincount(targets.reshape(B * N), C)
    stats = _tc_stats(x, t, B, C, NR, TNR=min(1024, NR))

    wpb = _NW // B
    out = pl.pallas_call(
        functools.partial(_combine_body, B=B, C=C, wpb=wpb),
        in_specs=[
            pl.BlockSpec((B, 2, C, 128), lambda: (0, 0, 0, 0)),
            pl.BlockSpec((_NW, C, _LANES), lambda: (0, 0, 0)),
        ],
        out_specs=pl.BlockSpec(memory_space=pltpu.SMEM),
        out_shape=jax.ShapeDtypeStruct((1,), jnp.float32),
    )(stats, counts)
    return out[0]


# speedup vs baseline: 2.2687x; 2.2687x over previous
import functools

import jax
import jax.numpy as jnp
from jax import lax
from jax.experimental import pallas as pl
from jax.experimental.pallas import tpu as pltpu
from jax.experimental.pallas import tpu_sc as plsc

SMOOTH = 1.0
IGNORE_INDEX = 0

_NW = 32
_LANES = 16


def _sc_bincount_body(t_hbm, out_hbm, tloc, tbl, *, chunk, C):
    cid = lax.axis_index("c")
    sid = lax.axis_index("s")
    wid = cid * 16 + sid
    base = wid * chunk

    pltpu.sync_copy(t_hbm.at[pl.ds(base, chunk)], tloc)

    zi = jnp.zeros((_LANES,), jnp.int32)
    one_i = jnp.ones((_LANES,), jnp.int32)
    unroll = 64

    def step(i, carry):
        pk = [zi, zi, zi, zi]
        cnts = list(carry)
        for u in range(unroll):
            t = tloc[pl.ds((unroll * i + u) * _LANES, _LANES)]
            hi = t >> 2
            inc = one_i << ((t & 3) << 3)
            for k in range(4):
                pk[k] = pk[k] + jnp.where(hi == k, inc, zi)
        for c in range(C):
            cnts[c] = cnts[c] + ((pk[c >> 2] >> ((c & 3) * 8)) & 255)
        return tuple(cnts)

    cnts = lax.fori_loop(0, chunk // (unroll * _LANES), step,
                         tuple(zi for _ in range(C)))
    for c in range(C):
        tbl[pl.ds(c * _LANES, _LANES)] = cnts[c].astype(jnp.float32)
    pltpu.sync_copy(tbl, out_hbm.at[wid])


def _sc_bincount(targets_flat, C):
    total = targets_flat.shape[0]
    chunk = total // _NW
    mesh = plsc.VectorSubcoreMesh(core_axis_name="c", subcore_axis_name="s")
    body = functools.partial(_sc_bincount_body, chunk=chunk, C=C)
    k = pl.kernel(
        body,
        mesh=mesh,
        out_type=jax.ShapeDtypeStruct((_NW, C * _LANES), jnp.float32),
        scratch_types=[
            pltpu.VMEM((chunk,), jnp.int32),
            pltpu.VMEM((C * _LANES,), jnp.float32),
        ],
    )
    return k(targets_flat).reshape(_NW, C, _LANES)


def _stats_body(*refs, B, C, nchunk, tnr):
    x_refs = refs[:C]
    t_ref = refs[C]
    out_ref = refs[C + 1]
    acc_ref = refs[C + 2]

    b = pl.program_id(0)
    n = pl.program_id(1)

    @pl.when((b == 0) & (n == 0))
    def _init():
        acc_ref[...] = jnp.zeros_like(acc_ref)

    rows = [r[0, 0] for r in x_refs]
    tf = t_ref[0, 0].astype(jnp.float32)

    es = [jnp.exp(jnp.minimum(rows[c], 80.0)) for c in range(C)]
    s = es[0]
    for c in range(1, C):
        s = s + es[c]
    r = 1.0 / s

    ones_row = jnp.ones((1, tnr), jnp.float32)

    def mxsum(v):
        return jax.lax.dot(ones_row, v, preferred_element_type=jnp.float32)

    zero = jnp.zeros((), jnp.float32)
    for c in range(C):
        g = es[c] * r
        mask = tf == float(c)
        acc_ref[b, 0, c] += mxsum(jnp.where(mask, g, zero))
        acc_ref[b, 1, c] += mxsum(g)

    @pl.when((b == B - 1) & (n == nchunk - 1))
    def _finish():
        out_ref[...] = jnp.sum(acc_ref[...], axis=3)


def _tc_stats(x, t, B, C, NR, TNR):
    nchunk = NR // TNR
    body = functools.partial(_stats_body, B=B, C=C, nchunk=nchunk, tnr=TNR)

    def xspec(c):
        return pl.BlockSpec((1, 1, TNR, 128), lambda b, n: (b, c, n, 0))

    return pl.pallas_call(
        body,
        grid=(B, nchunk),
        in_specs=[xspec(c) for c in range(C)] + [
            pl.BlockSpec((1, 1, TNR, 128), lambda b, n: (b, 0, n, 0)),
        ],
        out_specs=pl.BlockSpec((B, 2, C, 128), lambda b, n: (0, 0, 0, 0)),
        out_shape=jax.ShapeDtypeStruct((B, 2, C, 128), jnp.float32),
        scratch_shapes=[pltpu.VMEM((B, 2, C, 1, 128), jnp.float32)],
    )(*([x] * C + [t]))


def _combine_body(stats_ref, cnt_ref, out_ref, *, B, C, wpb):
    stats = jnp.sum(stats_ref[...], axis=3)
    cmask = (jax.lax.broadcasted_iota(jnp.int32, (1, C), 1)
             != IGNORE_INDEX).astype(jnp.float32)
    dice_sum = jnp.zeros((), jnp.float32)
    for b in range(B):
        cnt_b = jnp.sum(cnt_ref[pl.ds(b * wpb, wpb)], axis=(0, 2)).reshape(1, C)
        inter = stats[b, 0].reshape(1, C)
        union = stats[b, 1].reshape(1, C) + cnt_b
        dice = (2.0 * inter + SMOOTH) / (union + SMOOTH)
        dice_sum = dice_sum + jnp.sum(dice * cmask)
    out_ref[0] = 1.0 - dice_sum / (B * (C - 1))


def kernel(logits, targets):
    B, C = logits.shape[0], logits.shape[1]
    N = targets.shape[1] * targets.shape[2] * targets.shape[3]
    NR = N // 128
    x = logits.astype(jnp.float32).reshape(B, C, NR, 128)
    t = targets.reshape(B, 1, NR, 128)

    counts = _sc_bincount(targets.reshape(B * N), C)
    stats = _tc_stats(x, t, B, C, NR, TNR=min(1024, NR))

    wpb = _NW // B
    out = pl.pallas_call(
        functools.partial(_combine_body, B=B, C=C, wpb=wpb),
        in_specs=[
            pl.BlockSpec((B, 2, C, 128), lambda: (0, 0, 0, 0)),
            pl.BlockSpec((_NW, C, _LANES), lambda: (0, 0, 0)),
        ],
        out_specs=pl.BlockSpec(memory_space=pltpu.SMEM),
        out_shape=jax.ShapeDtypeStruct((1,), jnp.float32),
    )(stats, counts)
    return out[0]


# speedup vs baseline: 2.8486x; 1.2556x over previous
import functools

import jax
import jax.numpy as jnp
from jax.experimental import pallas as pl
from jax.experimental.pallas import tpu as pltpu

SMOOTH = 1.0
IGNORE_INDEX = 0


def _dice_body(*refs, B, C, nchunk, tnr):
    x_refs = refs[:C]
    t_ref = refs[C]
    out_ref = refs[C + 1]
    acc_ref = refs[C + 2]

    b = pl.program_id(0)
    n = pl.program_id(1)

    @pl.when((b == 0) & (n == 0))
    def _init():
        acc_ref[...] = jnp.zeros_like(acc_ref)

    rows = [r[0, 0] for r in x_refs]
    tf = t_ref[0, 0].astype(jnp.float32)

    es = [jnp.exp(jnp.minimum(rows[c], 80.0)) for c in range(C)]
    s = es[0]
    for c in range(1, C):
        s = s + es[c]
    r = 1.0 / s

    ones_row = jnp.ones((1, tnr), jnp.float32)

    def mxsum(v):
        return jax.lax.dot(ones_row, v, preferred_element_type=jnp.float32)

    zero = jnp.zeros((), jnp.float32)
    one = jnp.ones((), jnp.float32)
    for c in range(C):
        g = es[c] * r
        mask = tf == float(c)
        acc_ref[b, 0, c] += mxsum(jnp.where(mask, g, zero))
        acc_ref[b, 1, c] += mxsum(g)
        acc_ref[b, 2, c] += mxsum(jnp.where(mask, one, zero))

    @pl.when((b == B - 1) & (n == nchunk - 1))
    def _finish():
        stats = jnp.sum(acc_ref[...], axis=(3, 4))
        inter_bc = stats[:, 0, :]
        union_bc = stats[:, 1, :] + stats[:, 2, :]
        dice = (2.0 * inter_bc + SMOOTH) / (union_bc + SMOOTH)
        cmask = (jax.lax.broadcasted_iota(jnp.int32, (1, C), 1)
                 != IGNORE_INDEX).astype(jnp.float32)
        mean_dice = jnp.sum(dice * cmask) / (B * (C - 1))
        out_ref[0] = 1.0 - mean_dice


def kernel(logits, targets):
    B, C = logits.shape[0], logits.shape[1]
    N = targets.shape[1] * targets.shape[2] * targets.shape[3]
    NR = N // 128
    x = logits.astype(jnp.float32).reshape(B, C, NR, 128)
    t = targets.reshape(B, 1, NR, 128)

    TNR = min(1024, NR)
    nchunk = NR // TNR

    body = functools.partial(_dice_body, B=B, C=C, nchunk=nchunk, tnr=TNR)

    def xspec(c):
        return pl.BlockSpec((1, 1, TNR, 128), lambda b, n: (b, c, n, 0))

    out = pl.pallas_call(
        body,
        grid=(B, nchunk),
        in_specs=[xspec(c) for c in range(C)] + [
            pl.BlockSpec((1, 1, TNR, 128), lambda b, n: (b, 0, n, 0)),
        ],
        out_specs=pl.BlockSpec(memory_space=pltpu.SMEM),
        out_shape=jax.ShapeDtypeStruct((1,), jnp.float32),
        scratch_shapes=[pltpu.VMEM((B, 3, C, 1, 128), jnp.float32)],
    )(*([x] * C + [t]))
    return out[0]


# speedup vs baseline: 2.8635x; 1.0052x over previous
import functools

import jax
import jax.numpy as jnp
from jax.experimental import pallas as pl
from jax.experimental.pallas import tpu as pltpu

SMOOTH = 1.0
IGNORE_INDEX = 0


def _dice_body(x_ref, t_ref, out_ref, acc_ref, *, B, C, nchunk, tnr):
    b = pl.program_id(0)
    n = pl.program_id(1)

    @pl.when((b == 0) & (n == 0))
    def _init():
        acc_ref[...] = jnp.zeros_like(acc_ref)

    rows = [x_ref[0, c] for c in range(C)]
    tf = t_ref[0, 0].astype(jnp.float32)

    es = [jnp.exp(jnp.minimum(rows[c], 80.0)) for c in range(C)]
    s = es[0]
    for c in range(1, C):
        s = s + es[c]
    r = 1.0 / s

    ones_row = jnp.ones((1, tnr), jnp.float32)

    def mxsum(v):
        return jax.lax.dot(ones_row, v, preferred_element_type=jnp.float32)

    zero = jnp.zeros((), jnp.float32)
    one = jnp.ones((), jnp.float32)
    for c in range(C):
        g = es[c] * r
        mask = tf == float(c)
        acc_ref[b, 0, c] += mxsum(jnp.where(mask, g, zero))
        acc_ref[b, 1, c] += mxsum(g)
        acc_ref[b, 2, c] += mxsum(jnp.where(mask, one, zero))

    @pl.when((b == B - 1) & (n == nchunk - 1))
    def _finish():
        stats = jnp.sum(acc_ref[...], axis=(3, 4))
        inter_bc = stats[:, 0, :]
        union_bc = stats[:, 1, :] + stats[:, 2, :]
        dice = (2.0 * inter_bc + SMOOTH) / (union_bc + SMOOTH)
        cmask = (jax.lax.broadcasted_iota(jnp.int32, (1, C), 1)
                 != IGNORE_INDEX).astype(jnp.float32)
        mean_dice = jnp.sum(dice * cmask) / (B * (C - 1))
        out_ref[0] = 1.0 - mean_dice


def kernel(logits, targets):
    B, C = logits.shape[0], logits.shape[1]
    N = targets.shape[1] * targets.shape[2] * targets.shape[3]
    NR = N // 128
    x = logits.astype(jnp.float32).reshape(B, C, NR, 128)
    t = targets.reshape(B, 1, NR, 128)

    TNR = min(1024, NR)
    nchunk = NR // TNR

    body = functools.partial(_dice_body, B=B, C=C, nchunk=nchunk, tnr=TNR)

    out = pl.pallas_call(
        body,
        grid=(B, nchunk),
        in_specs=[
            pl.BlockSpec((1, C, TNR, 128), lambda b, n: (b, 0, n, 0)),
            pl.BlockSpec((1, 1, TNR, 128), lambda b, n: (b, 0, n, 0)),
        ],
        out_specs=pl.BlockSpec(memory_space=pltpu.SMEM),
        out_shape=jax.ShapeDtypeStruct((1,), jnp.float32),
        scratch_shapes=[pltpu.VMEM((B, 3, C, 1, 128), jnp.float32)],
    )(x, t)
    return out[0]
